# baseline (speedup 1.0000x reference)
import functools

import jax
import jax.numpy as jnp
from jax import lax
from jax._src import core as _jax_core
from jax._src.pallas import core as _pallas_core
from jax.experimental import pallas as pl
from jax.experimental.pallas import tpu as pltpu
from jax.experimental.pallas import tpu_sc as plsc

VOCAB = 1000000
EMBED = 64
BATCH = 16384

_N_SC = 8192
_N_TC = BATCH - _N_SC
_TC_CHUNK = 256
_SC_HALF = 256


def _to_default_space(x):
  return _pallas_core.with_memory_space_constraint_p.bind(
      x, memory_space=_jax_core.MemorySpace.Device)


def _build_sc_kernel():
  info = plsc.get_sparse_core_info()
  nc, ns = info.num_cores, info.num_subcores
  nw = nc * ns
  b_per_w = _N_SC // nw
  n_halves = b_per_w // _SC_HALF

  mesh = plsc.VectorSubcoreMesh(core_axis_name="c", subcore_axis_name="s")

  @functools.partial(
      pl.kernel,
      mesh=mesh,
      out_type=(
          pltpu.HBM((_N_SC, EMBED), jnp.float32),
          pltpu.HBM((_N_SC, EMBED), jnp.float32),
      ),
      scratch_types=[
          pltpu.VMEM((b_per_w,), jnp.int32),
          pltpu.VMEM((b_per_w,), jnp.int32),
          pltpu.VMEM((_SC_HALF, EMBED), jnp.float32),
          pltpu.VMEM((_SC_HALF, EMBED), jnp.float32),
          pltpu.SemaphoreType.DMA,
          pltpu.SemaphoreType.DMA,
      ],
  )
  def sc_lookup(center_hbm, context_hbm, word_hbm, ctx_hbm,
                out_c, out_x, cidx_v, xidx_v, crows_v, xrows_v,
                sem_c, sem_x):
    wid = lax.axis_index("s") * nc + lax.axis_index("c")
    base = wid * b_per_w

    pltpu.sync_copy(center_hbm.at[pl.ds(_N_TC + base, b_per_w)], cidx_v)
    pltpu.sync_copy(context_hbm.at[pl.ds(_N_TC + base, b_per_w)], xidx_v)

    for half in range(n_halves):
      def group_body(g, _):
        cv = cidx_v[pl.ds(half * _SC_HALF + g * 16, 16)]
        xv = xidx_v[pl.ds(half * _SC_HALF + g * 16, 16)]
        for lane in range(16):
          pltpu.async_copy(word_hbm.at[pl.ds(cv[lane], 1)],
                           crows_v.at[pl.ds(g * 16 + lane, 1)], sem_c)
          pltpu.async_copy(ctx_hbm.at[pl.ds(xv[lane], 1)],
                           xrows_v.at[pl.ds(g * 16 + lane, 1)], sem_x)
        return 0

      lax.fori_loop(0, _SC_HALF // 16, group_body, 0)

      pltpu.make_async_copy(word_hbm.at[pl.ds(0, _SC_HALF)], crows_v,
                            sem_c).wait()
      pltpu.make_async_copy(ctx_hbm.at[pl.ds(0, _SC_HALF)], xrows_v,
                            sem_x).wait()

      pltpu.sync_copy(crows_v,
                      out_c.at[pl.ds(base + half * _SC_HALF, _SC_HALF)])
      pltpu.sync_copy(xrows_v,
                      out_x.at[pl.ds(base + half * _SC_HALF, _SC_HALF)])

  return sc_lookup


def _tc_body(cidx_s, xidx_s, word_any, ctx_any, out_c, out_x,
             cbuf, xbuf, sem_c, sem_x, sem_o):
  n_chunks = _N_TC // _TC_CHUNK

  def chunk_body(ci, _):
    row0 = ci * _TC_CHUNK

    def row_body(i, _):
      ci_idx = cidx_s[row0 + i]
      xi_idx = xidx_s[row0 + i]
      pltpu.make_async_copy(word_any.at[pl.ds(ci_idx, 1)],
                            cbuf.at[pl.ds(i, 1)], sem_c).start()
      pltpu.make_async_copy(ctx_any.at[pl.ds(xi_idx, 1)],
                            xbuf.at[pl.ds(i, 1)], sem_x).start()
      return 0

    lax.fori_loop(0, _TC_CHUNK, row_body, 0)

    pltpu.make_async_copy(word_any.at[pl.ds(0, _TC_CHUNK)], cbuf,
                          sem_c).wait()
    pltpu.make_async_copy(ctx_any.at[pl.ds(0, _TC_CHUNK)], xbuf,
                          sem_x).wait()

    copy_c = pltpu.make_async_copy(cbuf, out_c.at[pl.ds(row0, _TC_CHUNK)],
                                   sem_o)
    copy_c.start()
    copy_c.wait()
    copy_x = pltpu.make_async_copy(xbuf, out_x.at[pl.ds(row0, _TC_CHUNK)],
                                   sem_o)
    copy_x.start()
    copy_x.wait()
    return 0

  lax.fori_loop(0, n_chunks, chunk_body, 0)


def _build_tc_kernel():
  return pl.pallas_call(
      _tc_body,
      in_specs=[
          pl.BlockSpec(memory_space=pltpu.SMEM),
          pl.BlockSpec(memory_space=pltpu.SMEM),
          pl.BlockSpec(memory_space=pltpu.HBM),
          pl.BlockSpec(memory_space=pltpu.HBM),
      ],
      out_specs=[
          pl.BlockSpec(memory_space=pltpu.HBM),
          pl.BlockSpec(memory_space=pltpu.HBM),
      ],
      out_shape=[
          jax.ShapeDtypeStruct((_N_TC, EMBED), jnp.float32),
          jax.ShapeDtypeStruct((_N_TC, EMBED), jnp.float32),
      ],
      scratch_shapes=[
          pltpu.VMEM((_TC_CHUNK, EMBED), jnp.float32),
          pltpu.VMEM((_TC_CHUNK, EMBED), jnp.float32),
          pltpu.SemaphoreType.DMA,
          pltpu.SemaphoreType.DMA,
          pltpu.SemaphoreType.DMA,
      ],
  )


_sc_lookup = _build_sc_kernel()
_tc_lookup = _build_tc_kernel()


@jax.jit
def kernel(center, context, word_embeds, context_embeds):
  c32 = center.astype(jnp.int32)
  x32 = context.astype(jnp.int32)
  sc_c, sc_x = _sc_lookup(c32, x32, word_embeds, context_embeds)
  tc_c, tc_x = _tc_lookup(c32[:_N_TC], x32[:_N_TC],
                          word_embeds, context_embeds)
  out_c = jnp.concatenate([tc_c, _to_default_space(sc_c)], axis=0)
  out_x = jnp.concatenate([tc_x, _to_default_space(sc_x)], axis=0)
  return out_c, out_x
